# baseline (speedup 1.0000x reference)
import functools

import jax
import jax.numpy as jnp
from jax import lax
from jax.experimental import pallas as pl
from jax.experimental.pallas import tpu as pltpu
from jax.experimental.pallas import tpu_sc as plsc

B, L, D = 4096, 199, 64
NC, NS = 2, 16
NW = NC * NS
BPW = B // NW


def _sc_body(embs_hbm, sl_hbm, tbl_hbm, out_hbm, sl_v, tbl_v, buf_v):
    wid = lax.axis_index("s") * NC + lax.axis_index("c")
    base = wid * BPW
    pltpu.sync_copy(sl_hbm.at[pl.ds(base, BPW)], sl_v)
    pltpu.sync_copy(tbl_hbm, tbl_v)

    def row_group(g, _):
        nvec = sl_v[pl.ds(g * 16, 16)]
        for j in range(16):
            b = g * 16 + j
            pltpu.sync_copy(embs_hbm.at[pl.ds(base + b, 1)], buf_v)
            n = nvec[j]

            def add_l(l, _):
                for c in range(4):
                    sl16 = pl.ds(c * 16, 16)
                    buf_v[0, l, sl16] = buf_v[0, l, sl16] + tbl_v[l, sl16]
                return 0

            lax.fori_loop(0, n, add_l, 0)
            pltpu.sync_copy(buf_v, out_hbm.at[pl.ds(base + b, 1)])
        return 0

    lax.fori_loop(0, BPW // 16, row_group, 0)


def kernel(embs, seq_lengths, table):
    tbl = table[1:L + 1]
    sl = seq_lengths.astype(jnp.int32)
    mesh = plsc.VectorSubcoreMesh(core_axis_name="c", subcore_axis_name="s")
    k = functools.partial(
        pl.kernel,
        out_type=jax.ShapeDtypeStruct((B, L, D), jnp.float32),
        mesh=mesh,
        scratch_types=[
            pltpu.VMEM((BPW,), jnp.int32),
            pltpu.VMEM((L, D), jnp.float32),
            pltpu.VMEM((1, L, D), jnp.float32),
        ],
    )(_sc_body)
    return k(embs, sl, tbl)


# speedup vs baseline: 1.2269x; 1.2269x over previous
import functools

import jax
import jax.numpy as jnp
from jax import lax
from jax.experimental import pallas as pl
from jax.experimental.pallas import tpu as pltpu
from jax.experimental.pallas import tpu_sc as plsc

B, L, D = 4096, 199, 64
NC, NS = 2, 16
NW = NC * NS
BPW = B // NW
K = 2
NCHUNK = BPW // K


def _sc_body(embs_hbm, sl_hbm, tbl_hbm, out_hbm,
             sl_v, tbl_v, buf0, buf1, si0, si1, so0, so1):
    wid = lax.axis_index("s") * NC + lax.axis_index("c")
    base = wid * BPW
    pltpu.sync_copy(sl_hbm.at[pl.ds(base, BPW)], sl_v)
    pltpu.sync_copy(tbl_hbm, tbl_v)

    bufs = (buf0, buf1)
    isems = (si0, si1)
    osems = (so0, so1)

    def load(g, p):
        pltpu.make_async_copy(
            embs_hbm.at[pl.ds(base + g * K, K)], bufs[p], isems[p]).start()

    def store(g, p):
        pltpu.make_async_copy(
            bufs[p], out_hbm.at[pl.ds(base + g * K, K)], osems[p]).start()

    def compute(g, p):
        buf = bufs[p]
        off = min(g * K, BPW - 16)
        nvec = sl_v[pl.ds(off, 16)]
        for j in range(K):
            n = nvec[g * K - off + j]

            def add_l(l, _):
                for c in range(4):
                    s16 = pl.ds(c * 16, 16)
                    buf[j, l, s16] = buf[j, l, s16] + tbl_v[l, s16]
                return 0

            lax.fori_loop(0, n, add_l, 0)

    load(0, 0)
    for g in range(NCHUNK):
        p = g & 1
        q = p ^ 1
        pltpu.make_async_copy(
            embs_hbm.at[pl.ds(base + g * K, K)], bufs[p], isems[p]).wait()
        if g + 1 < NCHUNK:
            if g >= 1:
                pltpu.make_async_copy(
                    bufs[q], out_hbm.at[pl.ds(base + (g - 1) * K, K)],
                    osems[q]).wait()
            load(g + 1, q)
        compute(g, p)
        store(g, p)
    pltpu.make_async_copy(
        bufs[0], out_hbm.at[pl.ds(base + (NCHUNK - 2) * K, K)], osems[0]).wait()
    pltpu.make_async_copy(
        bufs[1], out_hbm.at[pl.ds(base + (NCHUNK - 1) * K, K)], osems[1]).wait()


def kernel(embs, seq_lengths, table):
    tbl = table[1:L + 1]
    sl = seq_lengths.astype(jnp.int32)
    mesh = plsc.VectorSubcoreMesh(core_axis_name="c", subcore_axis_name="s")
    k = functools.partial(
        pl.kernel,
        out_type=jax.ShapeDtypeStruct((B, L, D), jnp.float32),
        mesh=mesh,
        scratch_types=[
            pltpu.VMEM((BPW,), jnp.int32),
            pltpu.VMEM((L, D), jnp.float32),
            pltpu.VMEM((K, L, D), jnp.float32),
            pltpu.VMEM((K, L, D), jnp.float32),
            pltpu.SemaphoreType.DMA,
            pltpu.SemaphoreType.DMA,
            pltpu.SemaphoreType.DMA,
            pltpu.SemaphoreType.DMA,
        ],
    )(_sc_body)
    return k(embs, sl, tbl)
